# baseline (speedup 1.0000x reference)
import jax
import jax.numpy as jnp
from jax.experimental import pallas as pl
from jax.experimental.pallas import tpu as pltpu
from jax.experimental.pallas import tpu_sc as plsc

D_MODEL = 64
SCALE = 8.0
WINDOW = 128
LANES = 16


def kernel(x, table):
    b, s = x.shape
    n = b * s
    idx = x.reshape(1, n)
    assert n % WINDOW == 0
    grid = n // WINDOW

    mesh = plsc.VectorSubcoreMesh(core_axis_name="core", subcore_axis_name="subcore")

    @pl.kernel(
        out_type=jax.ShapeDtypeStruct((n, D_MODEL), table.dtype),
        mesh=mesh,
        compiler_params=pltpu.CompilerParams(use_tc_tiling_on_sc=False),
    )
    def emb_kernel(tab_hbm, i_hbm, o_hbm):
        def body(i_vmem, o_vmem):
            pltpu.sync_copy(tab_hbm.at[i_vmem.at[0]], o_vmem)

            @pl.loop(0, WINDOW)
            def _(r):
                for c in range(0, D_MODEL, LANES):
                    slc = (pl.ds(r, 1), pl.ds(c, LANES))
                    o_vmem.at[*slc][...] = o_vmem.at[*slc][...] * SCALE

        pltpu.emit_pipeline(
            body,
            grid=(grid,),
            in_specs=[pl.BlockSpec((1, WINDOW), index_map=lambda i: (0, i))],
            out_specs=[pl.BlockSpec((WINDOW, D_MODEL), index_map=lambda i: (i, 0))],
            core_axis_name=("core", "subcore"),
            dimension_semantics=(pltpu.PARALLEL,),
        )(i_hbm, o_hbm)

    out = emb_kernel(table, idx)
    return out.reshape(b, s, D_MODEL)


# speedup vs baseline: 1.0429x; 1.0429x over previous
import jax
import jax.numpy as jnp
from jax.experimental import pallas as pl
from jax.experimental.pallas import tpu as pltpu
from jax.experimental.pallas import tpu_sc as plsc

D_MODEL = 64
SCALE = 8.0
WINDOW = 512
LANES = 16


def kernel(x, table):
    b, s = x.shape
    n = b * s
    idx = x.reshape(1, n)
    assert n % WINDOW == 0
    grid = n // WINDOW

    mesh = plsc.VectorSubcoreMesh(core_axis_name="core", subcore_axis_name="subcore")

    @pl.kernel(
        out_type=jax.ShapeDtypeStruct((n, D_MODEL), table.dtype),
        mesh=mesh,
        compiler_params=pltpu.CompilerParams(use_tc_tiling_on_sc=False),
    )
    def emb_kernel(tab_hbm, i_hbm, o_hbm):
        def body(i_vmem, o_vmem):
            pltpu.sync_copy(tab_hbm.at[i_vmem.at[0]], o_vmem)

            @pl.loop(0, WINDOW)
            def _(r):
                for c in range(0, D_MODEL, LANES):
                    slc = (pl.ds(r, 1), pl.ds(c, LANES))
                    o_vmem.at[*slc][...] = o_vmem.at[*slc][...] * SCALE

        pltpu.emit_pipeline(
            body,
            grid=(grid,),
            in_specs=[pl.BlockSpec((1, WINDOW), index_map=lambda i: (0, i))],
            out_specs=[pl.BlockSpec((WINDOW, D_MODEL), index_map=lambda i: (i, 0))],
            core_axis_name=("core", "subcore"),
            dimension_semantics=(pltpu.PARALLEL,),
        )(i_hbm, o_hbm)

    out = emb_kernel(table, idx)
    return out.reshape(b, s, D_MODEL)


# speedup vs baseline: 1.4684x; 1.4080x over previous
import jax
import jax.numpy as jnp
from jax import lax
from jax.experimental import pallas as pl
from jax.experimental.pallas import tpu as pltpu
from jax.experimental.pallas import tpu_sc as plsc

D_MODEL = 64
SCALE = 8.0
LANES = 16

NC = 2
NS = 16
NW = NC * NS
C = 128
K = 4


def kernel(x, table):
    b, s = x.shape
    n = b * s
    idx = x.reshape(n)
    bt = n // NW
    nch = bt // C
    g_total = nch // K
    assert n % (NW * C * K) == 0 and g_total % 2 == 0

    mesh = plsc.VectorSubcoreMesh(core_axis_name="core", subcore_axis_name="subcore")

    @pl.kernel(
        out_type=jax.ShapeDtypeStruct((n, D_MODEL), table.dtype),
        mesh=mesh,
        compiler_params=pltpu.CompilerParams(use_tc_tiling_on_sc=False),
        scratch_types=(
            [pltpu.VMEM((bt,), jnp.int32)]
            + [pltpu.VMEM((C, D_MODEL), jnp.float32) for _ in range(2 * K)]
            + [pltpu.SemaphoreType.DMA for _ in range(4 * K + 1)]
        ),
    )
    def emb_kernel(tab_hbm, i_hbm, o_hbm, idx_v, *rest):
        bufs = rest[: 2 * K]
        gsems = rest[2 * K : 4 * K]
        osems = rest[4 * K : 6 * K]
        isem = rest[6 * K]
        wid = lax.axis_index("subcore") * NC + lax.axis_index("core")
        base = wid * bt
        pltpu.async_copy(i_hbm.at[pl.ds(base, bt)], idx_v, isem).wait()

        def gcopy(st, g, bb):
            c = g * K + bb
            return pltpu.make_async_copy(
                tab_hbm.at[idx_v.at[pl.ds(c * C, C)]], bufs[st + bb], gsems[st + bb]
            )

        def ocopy(st, g, bb):
            c = g * K + bb
            return pltpu.make_async_copy(
                bufs[st + bb], o_hbm.at[pl.ds(base + c * C, C)], osems[st + bb]
            )

        def scale(st, bb):
            buf = bufs[st + bb]

            @pl.loop(0, C)
            def _(r):
                for cc in range(0, D_MODEL, LANES):
                    sl = (pl.ds(r, 1), pl.ds(cc, LANES))
                    buf.at[*sl][...] = buf.at[*sl][...] * SCALE

        def process(st, g, bb):
            gcopy(st, g, bb).wait()
            scale(st, bb)
            ocopy(st, g, bb).start()

        for bb in range(K):
            gcopy(0, 0, bb).start()

        @pl.loop(0, g_total, step=2)
        def _(g):
            process(0, g, 0)
            process(0, g, 1)

            @pl.when(g > 0)
            def _():
                for bb in range(K):
                    ocopy(K, g - 1, bb).wait()

            for bb in range(K):
                gcopy(K, g + 1, bb).start()
            process(0, g, 2)
            process(0, g, 3)

            process(K, g + 1, 0)
            process(K, g + 1, 1)

            @pl.when(g + 2 < g_total)
            def _():
                for bb in range(K):
                    ocopy(0, g, bb).wait()
                for bb in range(K):
                    gcopy(0, g + 2, bb).start()

            process(K, g + 1, 2)
            process(K, g + 1, 3)

        for bb in range(K):
            ocopy(0, g_total - 2, bb).wait()
        for bb in range(K):
            ocopy(K, g_total - 1, bb).wait()

    out = emb_kernel(table, idx)
    return out.reshape(b, s, D_MODEL)


# speedup vs baseline: 1.8824x; 1.2819x over previous
import jax
import jax.numpy as jnp
from jax import lax
from jax.experimental import pallas as pl
from jax.experimental.pallas import tpu as pltpu
from jax.experimental.pallas import tpu_sc as plsc

D_MODEL = 64
SCALE = 8.0
LANES = 16

NC = 2
NS = 16
NW = NC * NS
C = 64
K = 4
SPAD = 56


def kernel(x, table):
    b, s = x.shape
    xp = jnp.concatenate([x, x[:, s - 6 :]], axis=1)
    n = b * SPAD
    idx = xp.reshape(n)
    bt = n // NW
    nch = bt // C
    g_total = nch // K
    assert n % (NW * C * K) == 0 and g_total % 2 == 0

    mesh = plsc.VectorSubcoreMesh(core_axis_name="core", subcore_axis_name="subcore")

    @pl.kernel(
        out_type=jax.ShapeDtypeStruct((n, 2 * D_MODEL), table.dtype),
        mesh=mesh,
        compiler_params=pltpu.CompilerParams(use_tc_tiling_on_sc=False),
        scratch_types=(
            [pltpu.VMEM((bt,), jnp.int32)]
            + [pltpu.VMEM((C, D_MODEL), jnp.float32) for _ in range(2 * K)]
            + [pltpu.SemaphoreType.DMA for _ in range(4 * K + 1)]
        ),
    )
    def emb_kernel(tab_hbm, i_hbm, o_hbm, idx_v, *rest):
        bufs = rest[: 2 * K]
        gsems = rest[2 * K : 4 * K]
        osems = rest[4 * K : 6 * K]
        isem = rest[6 * K]
        wid = lax.axis_index("subcore") * NC + lax.axis_index("core")
        base = wid * bt
        pltpu.async_copy(i_hbm.at[pl.ds(base, bt)], idx_v, isem).wait()

        def gcopy(st, g, bb):
            c = g * K + bb
            return pltpu.make_async_copy(
                tab_hbm.at[idx_v.at[pl.ds(c * C, C)]], bufs[st + bb], gsems[st + bb]
            )

        def ocopy(st, g, bb):
            c = g * K + bb
            return pltpu.make_async_copy(
                bufs[st + bb],
                o_hbm.at[pl.ds(base + c * C, C), pl.ds(0, D_MODEL)],
                osems[st + bb],
            )

        def scale(st, bb):
            buf = bufs[st + bb]

            @pl.loop(0, C)
            def _(r):
                for cc in range(0, D_MODEL, LANES):
                    sl = (pl.ds(r, 1), pl.ds(cc, LANES))
                    buf.at[*sl][...] = buf.at[*sl][...] * SCALE

        def process(st, g, bb):
            gcopy(st, g, bb).wait()
            scale(st, bb)
            ocopy(st, g, bb).start()

        for bb in range(K):
            gcopy(0, 0, bb).start()

        @pl.loop(0, g_total, step=2)
        def _(g):
            process(0, g, 0)
            process(0, g, 1)

            @pl.when(g > 0)
            def _():
                for bb in range(K):
                    ocopy(K, g - 1, bb).wait()

            for bb in range(K):
                gcopy(K, g + 1, bb).start()
            process(0, g, 2)
            process(0, g, 3)

            process(K, g + 1, 0)
            process(K, g + 1, 1)

            @pl.when(g + 2 < g_total)
            def _():
                for bb in range(K):
                    ocopy(0, g, bb).wait()
                for bb in range(K):
                    gcopy(0, g + 2, bb).start()

            process(K, g + 1, 2)
            process(K, g + 1, 3)

        for bb in range(K):
            ocopy(0, g_total - 2, bb).wait()
        for bb in range(K):
            ocopy(K, g_total - 1, bb).wait()

    out = emb_kernel(table, idx)
    return out.reshape(b, SPAD, 2 * D_MODEL)[:, :s, :D_MODEL]


# speedup vs baseline: 2.1356x; 1.1345x over previous
import jax
import jax.numpy as jnp
from jax import lax
from jax.experimental import pallas as pl
from jax.experimental.pallas import tpu as pltpu
from jax.experimental.pallas import tpu_sc as plsc

D_MODEL = 64
SCALE = 8.0

NC = 2
NS = 16
NW = NC * NS
C = 64
K = 4
SPAD = 56
TW = 4096


def _prep_table(table):
    v, d = table.shape
    tab_t = table.T

    def body(t_ref, o_ref):
        o_ref[:, :D_MODEL] = t_ref[...].T * SCALE
        o_ref[:, D_MODEL:] = jnp.zeros((TW, D_MODEL), jnp.float32)

    return pl.pallas_call(
        body,
        grid=((v + TW - 1) // TW,),
        in_specs=[pl.BlockSpec((d, TW), lambda j: (0, j))],
        out_specs=pl.BlockSpec((TW, 2 * D_MODEL), lambda j: (j, 0)),
        out_shape=jax.ShapeDtypeStruct((v, 2 * D_MODEL), jnp.float32),
        compiler_params=pltpu.CompilerParams(dimension_semantics=("arbitrary",)),
    )(tab_t)


def kernel(x, table):
    b, s = x.shape
    t128 = _prep_table(table)
    xp = jnp.concatenate([x, x[:, s - 6 :]], axis=1)
    n = b * SPAD
    idx = xp.reshape(n)
    bt = n // NW
    nch = bt // C
    g_total = nch // K
    assert n % (NW * C * K) == 0 and g_total % 2 == 0

    mesh = plsc.VectorSubcoreMesh(core_axis_name="core", subcore_axis_name="subcore")

    @pl.kernel(
        out_type=jax.ShapeDtypeStruct((n, 2 * D_MODEL), jnp.float32),
        mesh=mesh,
        compiler_params=pltpu.CompilerParams(use_tc_tiling_on_sc=False),
        scratch_types=(
            [pltpu.VMEM((bt,), jnp.int32)]
            + [pltpu.VMEM((C, 2 * D_MODEL), jnp.float32) for _ in range(2 * K)]
            + [pltpu.SemaphoreType.DMA for _ in range(4 * K + 1)]
        ),
    )
    def emb_kernel(tab_hbm, i_hbm, o_hbm, idx_v, *rest):
        bufs = rest[: 2 * K]
        gsems = rest[2 * K : 4 * K]
        osems = rest[4 * K : 6 * K]
        isem = rest[6 * K]
        wid = lax.axis_index("subcore") * NC + lax.axis_index("core")
        base = wid * bt
        pltpu.async_copy(i_hbm.at[pl.ds(base, bt)], idx_v, isem).wait()

        def gcopy(st, g, bb):
            c = g * K + bb
            return pltpu.make_async_copy(
                tab_hbm.at[idx_v.at[pl.ds(c * C, C)]], bufs[st + bb], gsems[st + bb]
            )

        def ocopy(st, g, bb):
            c = g * K + bb
            return pltpu.make_async_copy(
                bufs[st + bb], o_hbm.at[pl.ds(base + c * C, C)], osems[st + bb]
            )

        def process(st, g, bb):
            gcopy(st, g, bb).wait()
            ocopy(st, g, bb).start()

        for bb in range(K):
            gcopy(0, 0, bb).start()

        @pl.loop(0, g_total, step=2)
        def _(g):
            process(0, g, 0)
            process(0, g, 1)

            @pl.when(g > 0)
            def _():
                for bb in range(K):
                    ocopy(K, g - 1, bb).wait()

            for bb in range(K):
                gcopy(K, g + 1, bb).start()
            process(0, g, 2)
            process(0, g, 3)

            process(K, g + 1, 0)
            process(K, g + 1, 1)

            @pl.when(g + 2 < g_total)
            def _():
                for bb in range(K):
                    ocopy(0, g, bb).wait()
                for bb in range(K):
                    gcopy(0, g + 2, bb).start()

            process(K, g + 1, 2)
            process(K, g + 1, 3)

        for bb in range(K):
            ocopy(0, g_total - 2, bb).wait()
        for bb in range(K):
            ocopy(K, g_total - 1, bb).wait()

    out = emb_kernel(t128, idx)
    return out.reshape(b, SPAD, 2 * D_MODEL)[:, :s, :D_MODEL]


# speedup vs baseline: 2.2494x; 1.0533x over previous
import jax
import jax.numpy as jnp
from jax import lax
from jax.experimental import pallas as pl
from jax.experimental.pallas import tpu as pltpu
from jax.experimental.pallas import tpu_sc as plsc

D_MODEL = 64
SCALE = 8.0

NC = 2
NS = 16
NW = NC * NS
C = 64
K = 4
SPAD = 56
TW = 4096


def _prep_table(table):
    v, d = table.shape
    tab_t = table.T

    def body(t_ref, o_ref):
        o_ref[:, :D_MODEL] = t_ref[...].T * SCALE

    return pl.pallas_call(
        body,
        grid=((v + TW - 1) // TW,),
        in_specs=[pl.BlockSpec((d, TW), lambda j: (0, j))],
        out_specs=pl.BlockSpec((TW, 2 * D_MODEL), lambda j: (j, 0)),
        out_shape=jax.ShapeDtypeStruct((v, 2 * D_MODEL), jnp.float32),
        compiler_params=pltpu.CompilerParams(dimension_semantics=("parallel",)),
    )(tab_t)


def kernel(x, table):
    b, s = x.shape
    t128 = _prep_table(table)
    xp = jnp.concatenate([x, x[:, s - 6 :]], axis=1)
    n = b * SPAD
    idx = xp.reshape(n)
    bt = n // NW
    nch = bt // C
    g_total = nch // K
    assert n % (NW * C * K) == 0 and g_total % 2 == 0

    mesh = plsc.VectorSubcoreMesh(core_axis_name="core", subcore_axis_name="subcore")

    @pl.kernel(
        out_type=jax.ShapeDtypeStruct((n, 2 * D_MODEL), jnp.float32),
        mesh=mesh,
        compiler_params=pltpu.CompilerParams(use_tc_tiling_on_sc=False),
        scratch_types=(
            [pltpu.VMEM((bt,), jnp.int32)]
            + [pltpu.VMEM((C, 2 * D_MODEL), jnp.float32) for _ in range(2 * K)]
            + [pltpu.SemaphoreType.DMA for _ in range(4 * K + 1)]
        ),
    )
    def emb_kernel(tab_hbm, i_hbm, o_hbm, idx_v, *rest):
        bufs = rest[: 2 * K]
        gsems = rest[2 * K : 4 * K]
        osems = rest[4 * K : 6 * K]
        isem = rest[6 * K]
        wid = lax.axis_index("subcore") * NC + lax.axis_index("core")
        base = wid * bt
        pltpu.async_copy(i_hbm.at[pl.ds(base, bt)], idx_v, isem).wait()

        def gcopy(st, g, bb):
            c = g * K + bb
            return pltpu.make_async_copy(
                tab_hbm.at[idx_v.at[pl.ds(c * C, C)]], bufs[st + bb], gsems[st + bb]
            )

        def ocopy(st, g, bb):
            c = g * K + bb
            return pltpu.make_async_copy(
                bufs[st + bb].at[:, pl.ds(0, D_MODEL)],
                o_hbm.at[pl.ds(base + c * C, C), pl.ds(0, D_MODEL)],
                osems[st + bb],
            )

        def process(st, g, bb):
            gcopy(st, g, bb).wait()
            ocopy(st, g, bb).start()

        for bb in range(K):
            gcopy(0, 0, bb).start()

        @pl.loop(0, g_total, step=2)
        def _(g):
            process(0, g, 0)
            process(0, g, 1)

            @pl.when(g > 0)
            def _():
                for bb in range(K):
                    ocopy(K, g - 1, bb).wait()

            for bb in range(K):
                gcopy(K, g + 1, bb).start()
            process(0, g, 2)
            process(0, g, 3)

            process(K, g + 1, 0)
            process(K, g + 1, 1)

            @pl.when(g + 2 < g_total)
            def _():
                for bb in range(K):
                    ocopy(0, g, bb).wait()
                for bb in range(K):
                    gcopy(0, g + 2, bb).start()

            process(K, g + 1, 2)
            process(K, g + 1, 3)

        for bb in range(K):
            ocopy(0, g_total - 2, bb).wait()
        for bb in range(K):
            ocopy(K, g_total - 1, bb).wait()

    out = emb_kernel(t128, idx)
    return out.reshape(b, SPAD, 2 * D_MODEL)[:, :s, :D_MODEL]


# speedup vs baseline: 2.4680x; 1.0972x over previous
import jax
import jax.numpy as jnp
from jax import lax
from jax.experimental import pallas as pl
from jax.experimental.pallas import tpu as pltpu
from jax.experimental.pallas import tpu_sc as plsc

D_MODEL = 64
SCALE = 8.0

NC = 2
NS = 16
NW = NC * NS
C = 64
K = 4
SPAD = 56
TW = 8192


def _prep_table(table):
    v, d = table.shape
    tab_t = table.T

    def body(t_ref, o_ref):
        o_ref[:, :D_MODEL] = t_ref[...].T * SCALE

    return pl.pallas_call(
        body,
        grid=((v + TW - 1) // TW,),
        in_specs=[pl.BlockSpec((d, TW), lambda j: (0, j))],
        out_specs=pl.BlockSpec((TW, 2 * D_MODEL), lambda j: (j, 0)),
        out_shape=jax.ShapeDtypeStruct((v, 2 * D_MODEL), jnp.float32),
        compiler_params=pltpu.CompilerParams(dimension_semantics=("parallel",)),
    )(tab_t)


def kernel(x, table):
    b, s = x.shape
    t128 = _prep_table(table)
    xp = jnp.concatenate([x, x[:, s - 6 :]], axis=1)
    n = b * SPAD
    idx = xp.reshape(n)
    bt = n // NW
    nch = bt // C
    g_total = nch // K
    assert n % (NW * C * K) == 0 and g_total % 2 == 0

    mesh = plsc.VectorSubcoreMesh(core_axis_name="core", subcore_axis_name="subcore")

    @pl.kernel(
        out_type=jax.ShapeDtypeStruct((n, 2 * D_MODEL), jnp.float32),
        mesh=mesh,
        compiler_params=pltpu.CompilerParams(use_tc_tiling_on_sc=False),
        scratch_types=(
            [pltpu.VMEM((bt,), jnp.int32)]
            + [pltpu.VMEM((C, 2 * D_MODEL), jnp.float32) for _ in range(2 * K)]
            + [pltpu.SemaphoreType.DMA for _ in range(4 * K + 1)]
        ),
    )
    def emb_kernel(tab_hbm, i_hbm, o_hbm, idx_v, *rest):
        bufs = rest[: 2 * K]
        gsems = rest[2 * K : 4 * K]
        osems = rest[4 * K : 6 * K]
        isem = rest[6 * K]
        wid = lax.axis_index("subcore") * NC + lax.axis_index("core")
        base = wid * bt
        pltpu.async_copy(i_hbm.at[pl.ds(base, bt)], idx_v, isem).wait()

        def gcopy(st, g, bb):
            c = g * K + bb
            return pltpu.make_async_copy(
                tab_hbm.at[idx_v.at[pl.ds(c * C, C)]], bufs[st + bb], gsems[st + bb]
            )

        def ocopy(st, g, bb):
            c = g * K + bb
            return pltpu.make_async_copy(
                bufs[st + bb].at[:, pl.ds(0, D_MODEL)],
                o_hbm.at[pl.ds(base + c * C, C), pl.ds(0, D_MODEL)],
                osems[st + bb],
            )

        def process(st, g, bb):
            gcopy(st, g, bb).wait()
            ocopy(st, g, bb).start()

        for bb in range(K):
            gcopy(0, 0, bb).start()

        @pl.loop(0, g_total, step=2)
        def _(g):
            process(0, g, 0)
            process(0, g, 1)

            @pl.when(g > 0)
            def _():
                for bb in range(K):
                    ocopy(K, g - 1, bb).wait()

            for bb in range(K):
                gcopy(K, g + 1, bb).start()
            process(0, g, 2)
            process(0, g, 3)

            process(K, g + 1, 0)
            process(K, g + 1, 1)

            @pl.when(g + 2 < g_total)
            def _():
                for bb in range(K):
                    ocopy(0, g, bb).wait()
                for bb in range(K):
                    gcopy(0, g + 2, bb).start()

            process(K, g + 1, 2)
            process(K, g + 1, 3)

        for bb in range(K):
            ocopy(0, g_total - 2, bb).wait()
        for bb in range(K):
            ocopy(K, g_total - 1, bb).wait()

    out = emb_kernel(t128, idx)
    return out.reshape(b, SPAD, 2 * D_MODEL)[:, :s, :D_MODEL]


# speedup vs baseline: 2.5322x; 1.0260x over previous
import jax
import jax.numpy as jnp
from jax import lax
from jax.experimental import pallas as pl
from jax.experimental.pallas import tpu as pltpu
from jax.experimental.pallas import tpu_sc as plsc

D_MODEL = 64
SCALE = 8.0

NC = 2
NS = 16
NW = NC * NS
C = 64
K = 4
SPAD = 56
TW = 16384


def _prep_table(table):
    v, d = table.shape
    tab_t = table.T

    def body(t_ref, o_ref):
        o_ref[:, :D_MODEL] = t_ref[...].T * SCALE

    return pl.pallas_call(
        body,
        grid=((v + TW - 1) // TW,),
        in_specs=[pl.BlockSpec((d, TW), lambda j: (0, j))],
        out_specs=pl.BlockSpec((TW, 2 * D_MODEL), lambda j: (j, 0)),
        out_shape=jax.ShapeDtypeStruct((v, 2 * D_MODEL), jnp.float32),
        compiler_params=pltpu.CompilerParams(dimension_semantics=("parallel",)),
    )(tab_t)


def kernel(x, table):
    b, s = x.shape
    t128 = _prep_table(table)
    xp = jnp.concatenate([x, x[:, s - 6 :]], axis=1)
    n = b * SPAD
    idx = xp.reshape(n)
    bt = n // NW
    nch = bt // C
    g_total = nch // K
    assert n % (NW * C * K) == 0 and g_total % 2 == 0

    mesh = plsc.VectorSubcoreMesh(core_axis_name="core", subcore_axis_name="subcore")

    @pl.kernel(
        out_type=jax.ShapeDtypeStruct((n, 2 * D_MODEL), jnp.float32),
        mesh=mesh,
        compiler_params=pltpu.CompilerParams(use_tc_tiling_on_sc=False),
        scratch_types=(
            [pltpu.VMEM((bt,), jnp.int32)]
            + [pltpu.VMEM((C, 2 * D_MODEL), jnp.float32) for _ in range(2 * K)]
            + [pltpu.SemaphoreType.DMA for _ in range(4 * K + 1)]
        ),
    )
    def emb_kernel(tab_hbm, i_hbm, o_hbm, idx_v, *rest):
        bufs = rest[: 2 * K]
        gsems = rest[2 * K : 4 * K]
        osems = rest[4 * K : 6 * K]
        isem = rest[6 * K]
        wid = lax.axis_index("subcore") * NC + lax.axis_index("core")
        base = wid * bt
        pltpu.async_copy(i_hbm.at[pl.ds(base, bt)], idx_v, isem).wait()

        def gcopy(st, g, bb):
            c = g * K + bb
            return pltpu.make_async_copy(
                tab_hbm.at[idx_v.at[pl.ds(c * C, C)]], bufs[st + bb], gsems[st + bb]
            )

        def ocopy(st, g, bb):
            c = g * K + bb
            return pltpu.make_async_copy(
                bufs[st + bb].at[:, pl.ds(0, D_MODEL)],
                o_hbm.at[pl.ds(base + c * C, C), pl.ds(0, D_MODEL)],
                osems[st + bb],
            )

        def process(st, g, bb):
            gcopy(st, g, bb).wait()
            ocopy(st, g, bb).start()

        for bb in range(K):
            gcopy(0, 0, bb).start()

        @pl.loop(0, g_total, step=2)
        def _(g):
            process(0, g, 0)
            process(0, g, 1)

            @pl.when(g > 0)
            def _():
                for bb in range(K):
                    ocopy(K, g - 1, bb).wait()

            for bb in range(K):
                gcopy(K, g + 1, bb).start()
            process(0, g, 2)
            process(0, g, 3)

            process(K, g + 1, 0)
            process(K, g + 1, 1)

            @pl.when(g + 2 < g_total)
            def _():
                for bb in range(K):
                    ocopy(0, g, bb).wait()
                for bb in range(K):
                    gcopy(0, g + 2, bb).start()

            process(K, g + 1, 2)
            process(K, g + 1, 3)

        for bb in range(K):
            ocopy(0, g_total - 2, bb).wait()
        for bb in range(K):
            ocopy(K, g_total - 1, bb).wait()

    out = emb_kernel(t128, idx)
    return out.reshape(b, SPAD, 2 * D_MODEL)[:, :s, :D_MODEL]


# speedup vs baseline: 2.5550x; 1.0090x over previous
import jax
import jax.numpy as jnp
from jax import lax
from jax.experimental import pallas as pl
from jax.experimental.pallas import tpu as pltpu
from jax.experimental.pallas import tpu_sc as plsc

D_MODEL = 64
SCALE = 8.0

NC = 2
NS = 16
NW = NC * NS
C = 64
K = 4
SPAD = 56
TW = 32768


def _prep_table(table):
    v, d = table.shape
    tab_t = table.T

    def body(t_ref, o_ref):
        o_ref[:, :D_MODEL] = t_ref[...].T * SCALE

    return pl.pallas_call(
        body,
        grid=((v + TW - 1) // TW,),
        in_specs=[pl.BlockSpec((d, TW), lambda j: (0, j))],
        out_specs=pl.BlockSpec((TW, 2 * D_MODEL), lambda j: (j, 0)),
        out_shape=jax.ShapeDtypeStruct((v, 2 * D_MODEL), jnp.float32),
        compiler_params=pltpu.CompilerParams(dimension_semantics=("parallel",)),
    )(tab_t)


def kernel(x, table):
    b, s = x.shape
    t128 = _prep_table(table)
    xp = jnp.concatenate([x, x[:, s - 6 :]], axis=1)
    n = b * SPAD
    idx = xp.reshape(n)
    bt = n // NW
    nch = bt // C
    g_total = nch // K
    assert n % (NW * C * K) == 0 and g_total % 2 == 0

    mesh = plsc.VectorSubcoreMesh(core_axis_name="core", subcore_axis_name="subcore")

    @pl.kernel(
        out_type=jax.ShapeDtypeStruct((n, 2 * D_MODEL), jnp.float32),
        mesh=mesh,
        compiler_params=pltpu.CompilerParams(use_tc_tiling_on_sc=False),
        scratch_types=(
            [pltpu.VMEM((bt,), jnp.int32)]
            + [pltpu.VMEM((C, 2 * D_MODEL), jnp.float32) for _ in range(2 * K)]
            + [pltpu.SemaphoreType.DMA for _ in range(4 * K + 1)]
        ),
    )
    def emb_kernel(tab_hbm, i_hbm, o_hbm, idx_v, *rest):
        bufs = rest[: 2 * K]
        gsems = rest[2 * K : 4 * K]
        osems = rest[4 * K : 6 * K]
        isem = rest[6 * K]
        wid = lax.axis_index("subcore") * NC + lax.axis_index("core")
        base = wid * bt
        pltpu.async_copy(i_hbm.at[pl.ds(base, bt)], idx_v, isem).wait()

        def gcopy(st, g, bb):
            c = g * K + bb
            return pltpu.make_async_copy(
                tab_hbm.at[idx_v.at[pl.ds(c * C, C)]], bufs[st + bb], gsems[st + bb]
            )

        def ocopy(st, g, bb):
            c = g * K + bb
            return pltpu.make_async_copy(
                bufs[st + bb].at[:, pl.ds(0, D_MODEL)],
                o_hbm.at[pl.ds(base + c * C, C), pl.ds(0, D_MODEL)],
                osems[st + bb],
            )

        def process(st, g, bb):
            gcopy(st, g, bb).wait()
            ocopy(st, g, bb).start()

        for bb in range(K):
            gcopy(0, 0, bb).start()

        @pl.loop(0, g_total, step=2)
        def _(g):
            process(0, g, 0)
            process(0, g, 1)

            @pl.when(g > 0)
            def _():
                for bb in range(K):
                    ocopy(K, g - 1, bb).wait()

            for bb in range(K):
                gcopy(K, g + 1, bb).start()
            process(0, g, 2)
            process(0, g, 3)

            process(K, g + 1, 0)
            process(K, g + 1, 1)

            @pl.when(g + 2 < g_total)
            def _():
                for bb in range(K):
                    ocopy(0, g, bb).wait()
                for bb in range(K):
                    gcopy(0, g + 2, bb).start()

            process(K, g + 1, 2)
            process(K, g + 1, 3)

        for bb in range(K):
            ocopy(0, g_total - 2, bb).wait()
        for bb in range(K):
            ocopy(K, g_total - 1, bb).wait()

    out = emb_kernel(t128, idx)
    return out.reshape(b, SPAD, 2 * D_MODEL)[:, :s, :D_MODEL]
